# baseline (speedup 1.0000x reference)
import functools
import math

import jax
import jax.numpy as jnp
from jax import lax
from jax.experimental import pallas as pl
from jax.experimental.pallas import tpu as pltpu
from jax.experimental.pallas import tpu_sc as plsc

D_MODEL = 64
LANES = 16
NUM_CORES = 2
NUM_SUBCORES = 16
NUM_WORKERS = NUM_CORES * NUM_SUBCORES
GROUP = 128
CHUNK = 256


def _build_gather(seq, tokens):
    t_per_w = tokens // NUM_WORKERS
    halves = t_per_w // CHUNK
    n = seq * halves
    mesh = plsc.VectorSubcoreMesh(
        core_axis_name="c", subcore_axis_name="s",
        num_cores=NUM_CORES, num_subcores=NUM_SUBCORES)

    @functools.partial(
        pl.kernel,
        out_type=jax.ShapeDtypeStruct((seq * tokens // 2, 2 * D_MODEL),
                                      jnp.float32),
        mesh=mesh,
        scratch_types=[
            pltpu.VMEM((4, GROUP), jnp.int32),
            pltpu.VMEM((4, GROUP), jnp.int32),
            pltpu.VMEM((2, CHUNK, 128), jnp.float32),
            pltpu.VMEM((2, GROUP, 128), jnp.float32),
            [pltpu.SemaphoreType.DMA] * 2,
            [pltpu.SemaphoreType.DMA] * 2,
        ],
        compiler_params=pltpu.CompilerParams(needs_layout_passes=False),
    )
    def emb_kernel(tbl_hbm, xt_hbm, out_hbm, xv, idxv, rows, outb, gsem, osem):
        wid = lax.axis_index("s") * NUM_CORES + lax.axis_index("c")
        t0 = wid * t_per_w

        def pos(c):
            s = c // halves
            tch = t0 + (c % halves) * CHUNK
            return s, tch, pl.multiple_of((s * tokens + tch) // 2, GROUP)

        def idx_load(c, slot):
            s, tch, _ = pos(c)
            for j in range(CHUNK // GROUP):
                k = slot * 2 + j
                pltpu.sync_copy(
                    xt_hbm.at[s, pl.ds(tch + j * GROUP, GROUP)], xv.at[k])
                for m in range(GROUP // LANES):
                    sl = pl.ds(m * LANES, LANES)
                    idxv[k, sl] = xv[k, sl] >> 1

        def gather_start(slot):
            for j in range(CHUNK // GROUP):
                pltpu.async_copy(
                    tbl_hbm.at[idxv.at[slot * 2 + j]],
                    rows.at[slot, pl.ds(j * GROUP, GROUP)], gsem[slot])

        def gather_wait(slot):
            for j in range(CHUNK // GROUP):
                pltpu.make_async_copy(
                    tbl_hbm.at[idxv.at[slot * 2 + j]],
                    rows.at[slot, pl.ds(j * GROUP, GROUP)], gsem[slot]).wait()

        def store_start(c, slot):
            _, _, row0 = pos(c)
            pltpu.async_copy(
                outb.at[slot], out_hbm.at[pl.ds(row0, GROUP)], osem[slot])

        def store_wait(c, slot):
            _, _, row0 = pos(c)
            pltpu.make_async_copy(
                outb.at[slot], out_hbm.at[pl.ds(row0, GROUP)], osem[slot]).wait()

        def compute(slot):
            k0 = slot * 2

            @pl.loop(0, GROUP // LANES)
            def _g(g):
                for half in range(2):
                    offv = (xv[k0 + half, pl.ds(g * LANES, LANES)] & 1) \
                        * D_MODEL
                    for l in range(LANES):
                        j = g * LANES + l
                        off = offv[l]
                        for dd in range(D_MODEL // LANES):
                            src = pl.ds(off + dd * LANES, LANES)
                            dst = pl.ds(half * D_MODEL + dd * LANES, LANES)
                            outb[slot, j, dst] = \
                                rows[slot, half * GROUP + j, src] * 8.0

        idx_load(0, 0)
        gather_start(0)

        @pl.loop(0, n, step=2)
        def _chunks(c0):
            for b in range(2):
                c = c0 + b
                slot = b
                nslot = 1 - b

                @pl.when(c + 1 < n)
                def _():
                    idx_load(c + 1, nslot)
                    gather_start(nslot)

                gather_wait(slot)

                @pl.when(c >= 2)
                def _():
                    store_wait(c - 2, slot)

                compute(slot)
                store_start(c, slot)

        store_wait(n - 2, 0)
        store_wait(n - 1, 1)

    return emb_kernel


def _tc_transpose(seq, tokens):
    tb = 1024
    grid = (seq, tokens // tb)

    def body(in_ref, out_ref):
        x = in_ref[...]
        parts = []
        for q in range(4):
            blk = x[q * 128:(q + 1) * 128, :]
            parts.append(blk[:, :D_MODEL].T)
            parts.append(blk[:, D_MODEL:].T)
        out_ref[...] = jnp.concatenate(parts, axis=1)[None]

    return pl.pallas_call(
        body,
        grid=grid,
        in_specs=[pl.BlockSpec((tb // 2, 2 * D_MODEL),
                               lambda s, t: (s * (tokens // tb) + t, 0))],
        out_specs=pl.BlockSpec((1, D_MODEL, tb), lambda s, t: (s, 0, t)),
        out_shape=jax.ShapeDtypeStruct((seq, D_MODEL, tokens), jnp.float32),
    )


def kernel(x, table):
    s0, s1 = x.shape
    vocab, d = table.shape
    xt = x.astype(jnp.int32).T
    tbl = table.reshape(vocab // 2, 2 * d)
    emb2 = _build_gather(s1, s0)(tbl, xt)
    outt = _tc_transpose(s1, s0)(emb2)
    return outt.transpose(2, 0, 1)


# speedup vs baseline: 1.2602x; 1.2602x over previous
import functools
import math

import jax
import jax.numpy as jnp
from jax import lax
from jax.experimental import pallas as pl
from jax.experimental.pallas import tpu as pltpu
from jax.experimental.pallas import tpu_sc as plsc

D_MODEL = 64
LANES = 16
NUM_CORES = 2
NUM_SUBCORES = 16
NUM_WORKERS = NUM_CORES * NUM_SUBCORES
GROUP = 128
CHUNK = 256


def _build_gather(seq, tokens):
    t_per_w = tokens // NUM_WORKERS
    halves = t_per_w // CHUNK
    n = seq * halves
    mesh = plsc.VectorSubcoreMesh(
        core_axis_name="c", subcore_axis_name="s",
        num_cores=NUM_CORES, num_subcores=NUM_SUBCORES)

    @functools.partial(
        pl.kernel,
        out_type=jax.ShapeDtypeStruct((seq * tokens, 2 * D_MODEL),
                                      jnp.float32),
        mesh=mesh,
        scratch_types=[
            pltpu.VMEM((4, GROUP), jnp.int32),
            pltpu.VMEM((2, CHUNK, 128), jnp.float32),
            [pltpu.SemaphoreType.DMA] * 2,
            [pltpu.SemaphoreType.DMA] * 2,
        ],
        compiler_params=pltpu.CompilerParams(needs_layout_passes=False),
    )
    def emb_kernel(tbl_hbm, xt_hbm, out_hbm, xv, rows, gsem, osem):
        wid = lax.axis_index("s") * NUM_CORES + lax.axis_index("c")
        t0 = wid * t_per_w

        def pos(c):
            s = c // halves
            tch = t0 + (c % halves) * CHUNK
            return s, tch, pl.multiple_of(s * tokens + tch, CHUNK)

        def idx_load(c, slot):
            s, tch, _ = pos(c)
            for j in range(CHUNK // GROUP):
                pltpu.sync_copy(
                    xt_hbm.at[s, pl.ds(tch + j * GROUP, GROUP)],
                    xv.at[slot * 2 + j])

        def gather_start(slot):
            for j in range(CHUNK // GROUP):
                pltpu.async_copy(
                    tbl_hbm.at[xv.at[slot * 2 + j]],
                    rows.at[slot, pl.ds(j * GROUP, GROUP)], gsem[slot])

        def gather_wait(slot):
            for j in range(CHUNK // GROUP):
                pltpu.make_async_copy(
                    tbl_hbm.at[xv.at[slot * 2 + j]],
                    rows.at[slot, pl.ds(j * GROUP, GROUP)], gsem[slot]).wait()

        def store_start(c, slot):
            _, _, row0 = pos(c)
            pltpu.async_copy(
                rows.at[slot], out_hbm.at[pl.ds(row0, CHUNK)], osem[slot])

        def store_wait(c, slot):
            _, _, row0 = pos(c)
            pltpu.make_async_copy(
                rows.at[slot], out_hbm.at[pl.ds(row0, CHUNK)], osem[slot]).wait()

        idx_load(0, 0)
        gather_start(0)

        @pl.loop(0, n, step=2)
        def _chunks(c0):
            for b in range(2):
                c = c0 + b
                slot = b
                nslot = 1 - b

                @pl.when(c + 1 < n)
                def _():
                    idx_load(c + 1, nslot)

                    @pl.when(c >= 1)
                    def _():
                        store_wait(c - 1, nslot)

                    gather_start(nslot)

                gather_wait(slot)
                store_start(c, slot)

        store_wait(n - 2, 0)
        store_wait(n - 1, 1)

    return emb_kernel


def _tc_transpose(seq, tokens):
    tb = 1024
    grid = (seq, tokens // tb)

    def body(in_ref, out_ref):
        blk = in_ref[...]
        out_ref[...] = (blk[:, :D_MODEL].T * 8.0)[None]

    return pl.pallas_call(
        body,
        grid=grid,
        in_specs=[
            pl.BlockSpec((tb, 2 * D_MODEL),
                         lambda s, t: (s * (tokens // tb) + t, 0)),
        ],
        out_specs=pl.BlockSpec((1, D_MODEL, tb), lambda s, t: (s, 0, t)),
        out_shape=jax.ShapeDtypeStruct((seq, D_MODEL, tokens), jnp.float32),
    )


def kernel(x, table):
    s0, s1 = x.shape
    vocab, d = table.shape
    xt = x.astype(jnp.int32).T
    tbl = jnp.pad(table, ((0, 0), (0, d)))
    emb2 = _build_gather(s1, s0)(tbl, xt)
    outt = _tc_transpose(s1, s0)(emb2)
    return outt.transpose(2, 0, 1)
